# baseline (speedup 1.0000x reference)
import functools

import jax
import jax.numpy as jnp
from jax import lax
from jax.experimental import pallas as pl
from jax.experimental.pallas import tpu as pltpu
from jax.experimental.pallas import tpu_sc as plsc

B = 4096
L = 50
TRC = 8192
NTBLK = 13
NTAB = NTBLK * TRC
CHAR_D = 112
DIST_D = 16
OUT_D = CHAR_D + DIST_D
N_ROWS = B * L
NDIST = 201
NC = 2
NS = 16
NW = NC * NS
ROWS_PER_W = N_ROWS // NW
BATCH_PER_W = B // NW
G = 128
NCHUNK = L
NBUF = 5
PFD = 2
KITER = NCHUNK // NBUF


def _concat_embed_sc(x_hbm, d_hbm, char_hbm, dist_hbm, out_hbm,
                     xi_v, dvi_v, dexp_v, *bufs):
    orow = bufs[0:NBUF]
    cg = bufs[NBUF:2 * NBUF]
    cs = bufs[2 * NBUF:3 * NBUF]

    wid = lax.axis_index("s") * NC + lax.axis_index("c")
    base = wid * ROWS_PER_W
    bblk = wid * BATCH_PER_W
    pltpu.sync_copy(x_hbm.at[pl.ds(base, ROWS_PER_W)], xi_v)
    pltpu.sync_copy(d_hbm.at[pl.ds(bblk, BATCH_PER_W)], dvi_v)
    pltpu.async_copy(dist_hbm.at[dvi_v], dexp_v, cg[0]).wait()

    def issue_gather(g, b):
        pltpu.async_copy(char_hbm.at[xi_v.at[pl.ds(g * G, G)]], orow[b], cg[b])

    def wait_gather(b):
        pltpu.make_async_copy(char_hbm.at[pl.ds(0, G)], orow[b], cg[b]).wait()

    def issue_store(g, b):
        pltpu.async_copy(orow[b], out_hbm.at[g, pl.ds(bblk, G)], cs[b])

    def wait_store(b):
        pltpu.make_async_copy(orow[b], out_hbm.at[0, pl.ds(bblk, G)], cs[b]).wait()

    def fill_dist(b):
        ob = orow[b]

        def fb(i, carry):
            for j in range(4):
                r = i * 4 + j
                ob[r, pl.ds(CHAR_D, DIST_D)] = dexp_v[r, pl.ds(0, DIST_D)]
            return carry

        lax.fori_loop(0, G // 4, fb, 0)

    for b in range(PFD):
        issue_gather(b, b)

    def body(k, carry):
        for b in range(NBUF):
            g = k * NBUF + b
            wait_gather(b)
            fill_dist(b)
            issue_store(g, b)
            b3 = (b + PFD) % NBUF
            g3 = g + PFD
            if b + PFD < NBUF:
                @pl.when(k >= 1)
                def _():
                    wait_store(b3)
                    issue_gather(g3, b3)

                @pl.when(k == 0)
                def _():
                    issue_gather(g3, b3)
            else:
                @pl.when(k < KITER - 1)
                def _():
                    wait_store(b3)
                    issue_gather(g3, b3)
        return carry

    lax.fori_loop(0, KITER, body, 0)

    for b in range(NBUF):
        wait_store(b)


def _tr_body(in_ref, out_ref):
    blk = in_ref[...]
    out_ref[...] = jnp.pad(jnp.swapaxes(blk, 0, 1), ((0, 0), (0, DIST_D)))


@jax.jit
def _transpose_pad(charT):
    return pl.pallas_call(
        _tr_body,
        grid=(NTBLK,),
        in_specs=[pl.BlockSpec((CHAR_D, TRC), lambda i: (0, i))],
        out_specs=pl.BlockSpec((TRC, OUT_D), lambda i: (i, 0)),
        out_shape=jax.ShapeDtypeStruct((NTAB, OUT_D), jnp.float32),
    )(charT)


@jax.jit
def _run(xarr, d, char128, dist128):
    mesh = plsc.VectorSubcoreMesh(core_axis_name="c", subcore_axis_name="s")
    scratch = [
        pltpu.VMEM((ROWS_PER_W,), jnp.int32),
        pltpu.VMEM((BATCH_PER_W,), jnp.int32),
        pltpu.VMEM((BATCH_PER_W, OUT_D), jnp.float32),
    ]
    scratch += [pltpu.VMEM((G, OUT_D), jnp.float32) for _ in range(NBUF)]
    scratch += [pltpu.SemaphoreType.DMA for _ in range(2 * NBUF)]
    f = functools.partial(
        pl.kernel,
        mesh=mesh,
        out_type=jax.ShapeDtypeStruct((L, B, OUT_D), jnp.float32),
        scratch_types=scratch,
    )(_concat_embed_sc)
    return f(xarr, d, char128, dist128)


def kernel(x, d, char_table, dist_table):
    xarr = x.T.reshape(L, NW, BATCH_PER_W).swapaxes(0, 1).reshape(N_ROWS)
    char128 = _transpose_pad(char_table.T)
    dist128 = jnp.pad(dist_table, ((0, 0), (0, CHAR_D)))
    out_t = _run(xarr, d, char128, dist128)
    return jnp.swapaxes(out_t, 0, 1)


# speedup vs baseline: 1.0058x; 1.0058x over previous
import functools

import jax
import jax.numpy as jnp
from jax import lax
from jax.experimental import pallas as pl
from jax.experimental.pallas import tpu as pltpu
from jax.experimental.pallas import tpu_sc as plsc

B = 4096
L = 50
TRC = 8192
NTBLK = 13
NTAB = NTBLK * TRC
CHAR_D = 112
DIST_D = 16
OUT_D = CHAR_D + DIST_D
N_ROWS = B * L
NC = 2
NS = 16
NW = NC * NS
ROWS_PER_W = N_ROWS // NW
BATCH_PER_W = B // NW
G = 128
NCHUNK = L
NBUF = 5
PFD = 3
KITER = NCHUNK // NBUF


def _concat_embed_sc(x_hbm, d_hbm, char_hbm, dist_hbm, out_hbm,
                     xi_v, dvi_v, dexp_v, *bufs):
    orow = bufs[0:NBUF]
    cg = bufs[NBUF:2 * NBUF]
    cs = bufs[2 * NBUF:3 * NBUF]

    wid = lax.axis_index("s") * NC + lax.axis_index("c")
    base = wid * ROWS_PER_W
    bblk = wid * BATCH_PER_W
    pltpu.sync_copy(x_hbm.at[pl.ds(base, ROWS_PER_W)], xi_v)
    pltpu.sync_copy(d_hbm.at[pl.ds(bblk, BATCH_PER_W)], dvi_v)
    pltpu.async_copy(dist_hbm.at[dvi_v], dexp_v, cg[0]).wait()

    def issue_gather(g, b):
        pltpu.async_copy(char_hbm.at[xi_v.at[pl.ds(g * G, G)]], orow[b], cg[b])

    def wait_gather(b):
        pltpu.make_async_copy(char_hbm.at[pl.ds(0, G)], orow[b], cg[b]).wait()

    def issue_store(g, b):
        pltpu.async_copy(orow[b], out_hbm.at[g, pl.ds(bblk, G)], cs[b])

    def wait_store(b):
        pltpu.make_async_copy(orow[b], out_hbm.at[0, pl.ds(bblk, G)], cs[b]).wait()

    def fill_dist(b):
        ob = orow[b]

        def fb(i, carry):
            for j in range(4):
                r = i * 4 + j
                ob[r, pl.ds(CHAR_D, DIST_D)] = dexp_v[r, pl.ds(0, DIST_D)]
            return carry

        lax.fori_loop(0, G // 4, fb, 0)

    for b in range(PFD):
        issue_gather(b, b)

    def body(k, carry):
        for b in range(NBUF):
            g = k * NBUF + b
            wait_gather(b)
            fill_dist(b)
            issue_store(g, b)
            b3 = (b + PFD) % NBUF
            g3 = g + PFD
            if b + PFD < NBUF:
                @pl.when(k >= 1)
                def _():
                    wait_store(b3)
                    issue_gather(g3, b3)

                @pl.when(k == 0)
                def _():
                    issue_gather(g3, b3)
            else:
                @pl.when(k < KITER - 1)
                def _():
                    wait_store(b3)
                    issue_gather(g3, b3)
        return carry

    lax.fori_loop(0, KITER, body, 0)

    for b in range(NBUF):
        wait_store(b)


def _tr_body(in_ref, out_ref):
    blk = in_ref[...]
    out_ref[...] = jnp.pad(jnp.swapaxes(blk, 0, 1), ((0, 0), (0, DIST_D)))


@jax.jit
def _transpose_pad(charT):
    return pl.pallas_call(
        _tr_body,
        grid=(NTBLK,),
        in_specs=[pl.BlockSpec((CHAR_D, TRC), lambda i: (0, i))],
        out_specs=pl.BlockSpec((TRC, OUT_D), lambda i: (i, 0)),
        out_shape=jax.ShapeDtypeStruct((NTAB, OUT_D), jnp.float32),
    )(charT)


@jax.jit
def _run(xarr, d, char128, dist128):
    mesh = plsc.VectorSubcoreMesh(core_axis_name="c", subcore_axis_name="s")
    scratch = [
        pltpu.VMEM((ROWS_PER_W,), jnp.int32),
        pltpu.VMEM((BATCH_PER_W,), jnp.int32),
        pltpu.VMEM((BATCH_PER_W, OUT_D), jnp.float32),
    ]
    scratch += [pltpu.VMEM((G, OUT_D), jnp.float32) for _ in range(NBUF)]
    scratch += [pltpu.SemaphoreType.DMA for _ in range(2 * NBUF)]
    f = functools.partial(
        pl.kernel,
        mesh=mesh,
        out_type=jax.ShapeDtypeStruct((L, B, OUT_D), jnp.float32),
        scratch_types=scratch,
    )(_concat_embed_sc)
    return f(xarr, d, char128, dist128)


def kernel(x, d, char_table, dist_table):
    xarr = x.T.reshape(L, NW, BATCH_PER_W).swapaxes(0, 1).reshape(N_ROWS)
    char128 = _transpose_pad(char_table.T)
    dist128 = jnp.pad(dist_table, ((0, 0), (0, CHAR_D)))
    out_t = _run(xarr, d, char128, dist128)
    return jnp.swapaxes(out_t, 0, 1)
